# Initial kernel scaffold; baseline (speedup 1.0000x reference)
import jax
import jax.numpy as jnp
from jax.experimental import pallas as pl


def kernel(x, edge_index, att, W1_0, b1_0, W2_0, b2_0, W1_1, b1_1, W2_1, b2_1):
    raise NotImplementedError("write your pallas kernel here")


# speedup vs baseline: 3.4846x; 3.4846x over previous
import functools

import jax
import jax.numpy as jnp
from jax import lax
from jax.experimental import pallas as pl
from jax.experimental.pallas import tpu as pltpu
from jax.experimental.pallas import tpu_sc as plsc

N = 10000
N_PAD = 10240
E = 320000
NC, NS, L = 2, 16, 16
NW = NC * NS
EPW = E // NW
CHUNK = 80
NCHUNKS = EPW // CHUNK
ROWS_PER_TILE = N_PAD // NS
ROWS_BLK = 128


def _make_segsum(D):
    JV = D // L
    mesh = plsc.VectorSubcoreMesh(core_axis_name="c", subcore_axis_name="s")

    @functools.partial(
        pl.kernel,
        out_type=jax.ShapeDtypeStruct((NC, N_PAD, D), jnp.float32),
        mesh=mesh,
        compiler_params=pltpu.CompilerParams(use_tc_tiling_on_sc=False),
        scratch_types=[
            pltpu.VMEM((CHUNK,), jnp.int32),
            pltpu.VMEM((CHUNK,), jnp.int32),
            pltpu.VMEM((CHUNK,), jnp.float32),
            pltpu.VMEM((CHUNK, D), jnp.float32),
            pltpu.VMEM_SHARED((N_PAD, D), jnp.float32),
            pltpu.VMEM((ROWS_BLK, D), jnp.float32),
            pltpu.SemaphoreType.DMA,
        ],
    )
    def seg(h_hbm, src_hbm, dst_hbm, att_hbm, out_hbm,
            src_v, dst_v, att_v, rows_v, acc_sh, bounce_v, sem):
        c = lax.axis_index("c")
        s = lax.axis_index("s")
        wid = s * NC + c

        def zero_row(r, _):
            for j in range(JV):
                bounce_v[r, pl.ds(j * L, L)] = jnp.zeros((L,), jnp.float32)
            return 0
        lax.fori_loop(0, ROWS_BLK, zero_row, 0)
        for k in range(ROWS_PER_TILE // ROWS_BLK):
            pltpu.sync_copy(bounce_v,
                            acc_sh.at[pl.ds(s * ROWS_PER_TILE + k * ROWS_BLK,
                                            ROWS_BLK)])
        plsc.subcore_barrier()

        base = wid * EPW

        def chunk_body(i, _):
            off = base + i * CHUNK
            pltpu.sync_copy(src_hbm.at[pl.ds(off, CHUNK)], src_v)
            pltpu.sync_copy(dst_hbm.at[pl.ds(off, CHUNK)], dst_v)
            pltpu.sync_copy(att_hbm.at[pl.ds(off, CHUNK)], att_v)
            pltpu.async_copy(h_hbm.at[src_v], rows_v, sem).wait()

            def group_body(g, _):
                av = att_v[pl.ds(g * L, L)]
                for e16 in range(L):
                    a = av[e16]
                    r = g * L + e16
                    for j in range(JV):
                        sl = pl.ds(j * L, L)
                        rows_v[r, sl] = rows_v[r, sl] * a
                return 0
            lax.fori_loop(0, CHUNK // L, group_body, 0)

            pltpu.sync_copy(rows_v, acc_sh.at[dst_v], add=True)
            return 0
        lax.fori_loop(0, NCHUNKS, chunk_body, 0)
        plsc.subcore_barrier()

        for k in range(ROWS_PER_TILE // ROWS_BLK):
            r0 = s * ROWS_PER_TILE + k * ROWS_BLK
            pltpu.sync_copy(acc_sh.at[pl.ds(r0, ROWS_BLK)], bounce_v)
            pltpu.sync_copy(bounce_v, out_hbm.at[c, pl.ds(r0, ROWS_BLK)])

    return seg


_segsum128 = _make_segsum(128)
_segsum64 = _make_segsum(64)

_DENSE_BLK = 2000


def _dense_body(h_ref, p0_ref, p1_ref, w1_ref, b1_ref, w2_ref, b2_ref, o_ref):
    h = h_ref[...]
    nh = p0_ref[0] + p1_ref[0]
    z1 = jnp.dot(h + nh, w1_ref[...],
                 preferred_element_type=jnp.float32) + b1_ref[...]
    z2 = jnp.dot(h * nh, w2_ref[...],
                 preferred_element_type=jnp.float32) + b2_ref[...]
    o_ref[...] = (jnp.where(z1 >= 0, z1, 0.01 * z1)
                  + jnp.where(z2 >= 0, z2, 0.01 * z2))


def _dense(h, parts, w1, b1, w2, b2):
    din = h.shape[1]
    dout = w1.shape[1]
    b1 = b1.reshape(1, dout)
    b2 = b2.reshape(1, dout)
    return pl.pallas_call(
        _dense_body,
        grid=(N // _DENSE_BLK,),
        in_specs=[
            pl.BlockSpec((_DENSE_BLK, din), lambda i: (i, 0)),
            pl.BlockSpec((1, _DENSE_BLK, din), lambda i: (0, i, 0)),
            pl.BlockSpec((1, _DENSE_BLK, din), lambda i: (1, i, 0)),
            pl.BlockSpec((din, dout), lambda i: (0, 0)),
            pl.BlockSpec((1, dout), lambda i: (0, 0)),
            pl.BlockSpec((din, dout), lambda i: (0, 0)),
            pl.BlockSpec((1, dout), lambda i: (0, 0)),
        ],
        out_specs=pl.BlockSpec((_DENSE_BLK, dout), lambda i: (i, 0)),
        out_shape=jax.ShapeDtypeStruct((N, dout), jnp.float32),
    )(h, parts, parts, w1, b1, w2, b2)


def kernel(x, edge_index, att, W1_0, b1_0, W2_0, b2_0, W1_1, b1_1, W2_1, b2_1):
    src = edge_index[0]
    dst = edge_index[1]
    parts1 = _segsum128(x, src, dst, att)
    h1 = _dense(x, parts1, W1_0, b1_0, W2_0, b2_0)
    parts2 = _segsum64(h1, src, dst, att)
    out = _dense(h1, parts2, W1_1, b1_1, W2_1, b2_1)
    return out


# speedup vs baseline: 6.9750x; 2.0016x over previous
import functools

import jax
import jax.numpy as jnp
from jax import lax
from jax.experimental import pallas as pl
from jax.experimental.pallas import tpu as pltpu
from jax.experimental.pallas import tpu_sc as plsc

N = 10000
E = 320000
NC, NS, L = 2, 16, 16
NW = NC * NS
EPW = E // NW
CHUNK = 80
NCHUNKS = EPW // CHUNK
ROWS_PER_TILE = N // NS
_BLOCKS = [(k * 80, 80) for k in range(7)] + [(560, 65)]


def _make_segsum(D):
    JV = D // L
    mesh = plsc.VectorSubcoreMesh(core_axis_name="c", subcore_axis_name="s")

    @functools.partial(
        pl.kernel,
        out_type=jax.ShapeDtypeStruct((NC, N, D), jnp.float32),
        mesh=mesh,
        compiler_params=pltpu.CompilerParams(use_tc_tiling_on_sc=False),
        scratch_types=[
            pltpu.VMEM((4, 3, CHUNK), jnp.int32),
            pltpu.VMEM((CHUNK, D), jnp.float32),
            pltpu.VMEM((CHUNK, D), jnp.float32),
            pltpu.VMEM_SHARED((N, D), jnp.float32),
            pltpu.SemaphoreType.DMA,
            pltpu.SemaphoreType.DMA,
            pltpu.SemaphoreType.DMA,
            pltpu.SemaphoreType.DMA,
            pltpu.SemaphoreType.DMA,
            pltpu.SemaphoreType.DMA,
            pltpu.SemaphoreType.DMA,
            pltpu.SemaphoreType.DMA,
        ],
    )
    def seg(h_hbm, comb_hbm, out_hbm,
            ring_v, rows0_v, rows1_v, acc_sh,
            gsem0, gsem1, ssem0, ssem1, csem0, csem1, csem2, csem3):
        c = lax.axis_index("c")
        s = lax.axis_index("s")
        wid = s * NC + c
        rows = (rows0_v, rows1_v)
        gsem = (gsem0, gsem1)
        ssem = (ssem0, ssem1)
        csem = (csem0, csem1, csem2, csem3)

        def zero_row(r, _):
            for j in range(JV):
                rows0_v[r, pl.ds(j * L, L)] = jnp.zeros((L,), jnp.float32)
            return 0
        lax.fori_loop(0, CHUNK, zero_row, 0)
        for r0, nr in _BLOCKS:
            pltpu.sync_copy(
                rows0_v.at[pl.ds(0, nr)],
                acc_sh.at[pl.ds(s * ROWS_PER_TILE + r0, nr)])
        plsc.subcore_barrier()

        def start_comb(i, r):
            pltpu.async_copy(comb_hbm.at[wid, i], ring_v.at[r], csem[r])

        def wait_comb(i, r):
            pltpu.make_async_copy(comb_hbm.at[wid, i], ring_v.at[r],
                                  csem[r]).wait()

        def start_gather(r, b):
            pltpu.async_copy(h_hbm.at[ring_v.at[r, 0]], rows[b], gsem[b])

        def wait_gather(r, b):
            pltpu.make_async_copy(h_hbm.at[ring_v.at[r, 0]], rows[b],
                                  gsem[b]).wait()

        def start_scat(r, b):
            pltpu.async_copy(rows[b], acc_sh.at[ring_v.at[r, 1]], ssem[b],
                             add=True)

        def wait_scat(r, b):
            pltpu.make_async_copy(rows[b], acc_sh.at[ring_v.at[r, 1]],
                                  ssem[b]).wait()

        def scale(i, b, r):
            def group_body(g, _):
                av = lax.bitcast_convert_type(ring_v[r, 2, pl.ds(g * L, L)], jnp.float32)
                for e16 in range(L):
                    a = av[e16]
                    rr = g * L + e16
                    for j in range(JV):
                        sl = pl.ds(j * L, L)
                        rows[b][rr, sl] = rows[b][rr, sl] * a
                return 0
            lax.fori_loop(0, CHUNK // L, group_body, 0)

        start_comb(0, 0)
        start_comb(1, 1)
        start_comb(2, 2)
        wait_comb(0, 0)
        start_gather(0, 0)
        wait_gather(0, 0)
        wait_comb(1, 1)
        start_gather(1, 1)
        start_comb(3, 3)
        scale(0, 0, 0)
        start_scat(0, 0)

        def substep(i, r):
            b = r & 1
            wait_gather(r, b)
            wait_scat((r + 3) % 4, b ^ 1)
            start_comb(i + 3, (r + 3) % 4)
            wait_comb(i + 1, (r + 1) % 4)
            start_gather((r + 1) % 4, b ^ 1)
            scale(i, b, r)
            start_scat(r, b)

        def quad_body(k, _):
            i = 4 * k + 1
            substep(i, 1)
            substep(i + 1, 2)
            substep(i + 2, 3)
            substep(i + 3, 0)
            return 0
        lax.fori_loop(0, (NCHUNKS - 5) // 4, quad_body, 0)

        i = NCHUNKS - 4
        wait_gather(1, 1)
        wait_scat(0, 0)
        start_comb(i + 3, 0)
        wait_comb(i + 1, 2)
        start_gather(2, 0)
        scale(i, 1, 1)
        start_scat(1, 1)
        i = NCHUNKS - 3
        wait_gather(2, 0)
        wait_scat(1, 1)
        wait_comb(i + 1, 3)
        start_gather(3, 1)
        scale(i, 0, 2)
        start_scat(2, 0)
        i = NCHUNKS - 2
        wait_gather(3, 1)
        wait_scat(2, 0)
        wait_comb(i + 1, 0)
        start_gather(0, 0)
        scale(i, 1, 3)
        start_scat(3, 1)
        i = NCHUNKS - 1
        wait_gather(0, 0)
        wait_scat(3, 1)
        scale(i, 0, 0)
        start_scat(0, 0)
        wait_scat(0, 0)

        plsc.subcore_barrier()

        for r0, nr in _BLOCKS:
            a0 = s * ROWS_PER_TILE + r0
            pltpu.sync_copy(acc_sh.at[pl.ds(a0, nr)], rows0_v.at[pl.ds(0, nr)])
            pltpu.sync_copy(rows0_v.at[pl.ds(0, nr)],
                            out_hbm.at[c, pl.ds(a0, nr)])

    return seg


_segsum128 = _make_segsum(128)
_segsum64 = _make_segsum(64)

_DENSE_BLK = 2000


def _dense_body(h_ref, p0_ref, p1_ref, w1_ref, b1_ref, w2_ref, b2_ref, o_ref):
    h = h_ref[...]
    nh = p0_ref[0] + p1_ref[0]
    z1 = jnp.dot(h + nh, w1_ref[...],
                 preferred_element_type=jnp.float32) + b1_ref[...]
    z2 = jnp.dot(h * nh, w2_ref[...],
                 preferred_element_type=jnp.float32) + b2_ref[...]
    o_ref[...] = (jnp.where(z1 >= 0, z1, 0.01 * z1)
                  + jnp.where(z2 >= 0, z2, 0.01 * z2))


def _dense(h, parts, w1, b1, w2, b2):
    din = h.shape[1]
    dout = w1.shape[1]
    b1 = b1.reshape(1, dout)
    b2 = b2.reshape(1, dout)
    return pl.pallas_call(
        _dense_body,
        grid=(N // _DENSE_BLK,),
        in_specs=[
            pl.BlockSpec((_DENSE_BLK, din), lambda i: (i, 0)),
            pl.BlockSpec((1, _DENSE_BLK, din), lambda i: (0, i, 0)),
            pl.BlockSpec((1, _DENSE_BLK, din), lambda i: (1, i, 0)),
            pl.BlockSpec((din, dout), lambda i: (0, 0)),
            pl.BlockSpec((1, dout), lambda i: (0, 0)),
            pl.BlockSpec((din, dout), lambda i: (0, 0)),
            pl.BlockSpec((1, dout), lambda i: (0, 0)),
        ],
        out_specs=pl.BlockSpec((_DENSE_BLK, dout), lambda i: (i, 0)),
        out_shape=jax.ShapeDtypeStruct((N, dout), jnp.float32),
    )(h, parts, parts, w1, b1, w2, b2)


def kernel(x, edge_index, att, W1_0, b1_0, W2_0, b2_0, W1_1, b1_1, W2_1, b2_1):
    src = edge_index[0].reshape(NW, NCHUNKS, 1, CHUNK)
    dst = edge_index[1].reshape(NW, NCHUNKS, 1, CHUNK)
    att_i = lax.bitcast_convert_type(att, jnp.int32).reshape(
        NW, NCHUNKS, 1, CHUNK)
    comb = jnp.concatenate([src, dst, att_i], axis=2)
    parts1 = _segsum128(x, comb)
    h1 = _dense(x, parts1, W1_0, b1_0, W2_0, b2_0)
    parts2 = _segsum64(h1, comb)
    out = _dense(h1, parts2, W1_1, b1_1, W2_1, b2_1)
    return out


# speedup vs baseline: 8.1207x; 1.1643x over previous
import functools

import jax
import jax.numpy as jnp
from jax import lax
from jax.experimental import pallas as pl
from jax.experimental.pallas import tpu as pltpu
from jax.experimental.pallas import tpu_sc as plsc

N = 10000
E = 320000
NC, NS, L = 2, 16, 16
NW = NC * NS
EPW = E // NW
CHUNK = 80
NCHUNKS = EPW // CHUNK
ROWS_PER_TILE = N // NS
_BLOCKS = [(k * 80, 80) for k in range(7)] + [(560, 65)]


def _make_segsum(D):
    JV = D // L
    mesh = plsc.VectorSubcoreMesh(core_axis_name="c", subcore_axis_name="s")

    @functools.partial(
        pl.kernel,
        out_type=jax.ShapeDtypeStruct((NC, N, D), jnp.float32),
        mesh=mesh,
        compiler_params=pltpu.CompilerParams(use_tc_tiling_on_sc=False),
        scratch_types=[
            pltpu.VMEM((8, 3, CHUNK), jnp.int32),
            pltpu.VMEM((4, CHUNK, D), jnp.float32),
            pltpu.VMEM_SHARED((N, D), jnp.float32),
            [pltpu.SemaphoreType.DMA] * 4,
            [pltpu.SemaphoreType.DMA] * 4,
            [pltpu.SemaphoreType.DMA] * 8,
        ],
    )
    def seg(h_hbm, comb_hbm, out_hbm,
            ring_v, rows_v, acc_sh, gsem, ssem, csem):
        c = lax.axis_index("c")
        s = lax.axis_index("s")
        wid = s * NC + c

        def zero_row(r, _):
            for j in range(JV):
                rows_v[0, r, pl.ds(j * L, L)] = jnp.zeros((L,), jnp.float32)
            return 0
        lax.fori_loop(0, CHUNK, zero_row, 0)
        for r0, nr in _BLOCKS:
            pltpu.sync_copy(
                rows_v.at[0, pl.ds(0, nr)],
                acc_sh.at[pl.ds(s * ROWS_PER_TILE + r0, nr)])
        plsc.subcore_barrier()

        def start_comb(i, cs):
            pltpu.async_copy(comb_hbm.at[wid, i], ring_v.at[cs], csem[cs])

        def wait_comb(i, cs):
            pltpu.make_async_copy(comb_hbm.at[wid, i], ring_v.at[cs],
                                  csem[cs]).wait()

        def start_gather(cs, rs):
            pltpu.async_copy(h_hbm.at[ring_v.at[cs, 0]], rows_v.at[rs],
                             gsem[rs])

        def wait_gather(cs, rs):
            pltpu.make_async_copy(h_hbm.at[ring_v.at[cs, 0]], rows_v.at[rs],
                                  gsem[rs]).wait()

        def start_scat(cs, rs):
            pltpu.async_copy(rows_v.at[rs], acc_sh.at[ring_v.at[cs, 1]],
                             ssem[rs], add=True)

        def wait_scat(cs, rs):
            pltpu.make_async_copy(rows_v.at[rs], acc_sh.at[ring_v.at[cs, 1]],
                                  ssem[rs]).wait()

        def scale(i, cs, rs):
            def group_body(g, _):
                av = lax.bitcast_convert_type(
                    ring_v[cs, 2, pl.ds(g * L, L)], jnp.float32)
                for e16 in range(L):
                    a = av[e16]
                    rr = g * L + e16
                    for j in range(JV):
                        sl = pl.ds(j * L, L)
                        rows_v[rs, rr, sl] = rows_v[rs, rr, sl] * a
                return 0
            lax.fori_loop(0, CHUNK // L, group_body, 0)

        for k in range(6):
            start_comb(k, k)
        wait_comb(0, 0)
        start_gather(0, 0)
        wait_comb(1, 1)
        start_gather(1, 1)

        def octet_body(k, _):
            for p in range(8):
                i = 8 * k + p
                rs = p & 3

                @pl.when(i <= NCHUNKS - 1)
                def _():
                    wait_gather(p, rs)

                @pl.when(jnp.logical_and(i >= 2, i <= NCHUNKS + 1))
                def _():
                    wait_scat((p + 6) % 8, (p + 2) % 4)

                @pl.when(i <= NCHUNKS - 7)
                def _():
                    start_comb(i + 6, (p + 6) % 8)

                @pl.when(i <= NCHUNKS - 3)
                def _():
                    wait_comb(i + 2, (p + 2) % 8)
                    start_gather((p + 2) % 8, (p + 2) % 4)

                @pl.when(i <= NCHUNKS - 1)
                def _():
                    scale(i, p, rs)
                    start_scat(p, rs)
            return 0
        lax.fori_loop(0, (NCHUNKS + 9) // 8, octet_body, 0)

        plsc.subcore_barrier()

        for r0, nr in _BLOCKS:
            a0 = s * ROWS_PER_TILE + r0
            pltpu.sync_copy(acc_sh.at[pl.ds(a0, nr)],
                            rows_v.at[0, pl.ds(0, nr)])
            pltpu.sync_copy(rows_v.at[0, pl.ds(0, nr)],
                            out_hbm.at[c, pl.ds(a0, nr)])

    return seg


_segsum128 = _make_segsum(128)
_segsum64 = _make_segsum(64)

_DENSE_BLK = 2000


def _dense_body(h_ref, p0_ref, p1_ref, w1_ref, b1_ref, w2_ref, b2_ref, o_ref):
    h = h_ref[...]
    nh = p0_ref[0] + p1_ref[0]
    z1 = jnp.dot(h + nh, w1_ref[...],
                 preferred_element_type=jnp.float32) + b1_ref[...]
    z2 = jnp.dot(h * nh, w2_ref[...],
                 preferred_element_type=jnp.float32) + b2_ref[...]
    o_ref[...] = (jnp.where(z1 >= 0, z1, 0.01 * z1)
                  + jnp.where(z2 >= 0, z2, 0.01 * z2))


def _dense(h, parts, w1, b1, w2, b2):
    din = h.shape[1]
    dout = w1.shape[1]
    b1 = b1.reshape(1, dout)
    b2 = b2.reshape(1, dout)
    return pl.pallas_call(
        _dense_body,
        grid=(N // _DENSE_BLK,),
        in_specs=[
            pl.BlockSpec((_DENSE_BLK, din), lambda i: (i, 0)),
            pl.BlockSpec((1, _DENSE_BLK, din), lambda i: (0, i, 0)),
            pl.BlockSpec((1, _DENSE_BLK, din), lambda i: (1, i, 0)),
            pl.BlockSpec((din, dout), lambda i: (0, 0)),
            pl.BlockSpec((1, dout), lambda i: (0, 0)),
            pl.BlockSpec((din, dout), lambda i: (0, 0)),
            pl.BlockSpec((1, dout), lambda i: (0, 0)),
        ],
        out_specs=pl.BlockSpec((_DENSE_BLK, dout), lambda i: (i, 0)),
        out_shape=jax.ShapeDtypeStruct((N, dout), jnp.float32),
    )(h, parts, parts, w1, b1, w2, b2)


def kernel(x, edge_index, att, W1_0, b1_0, W2_0, b2_0, W1_1, b1_1, W2_1, b2_1):
    src = edge_index[0].reshape(NW, NCHUNKS, 1, CHUNK)
    dst = edge_index[1].reshape(NW, NCHUNKS, 1, CHUNK)
    att_i = lax.bitcast_convert_type(att, jnp.int32).reshape(
        NW, NCHUNKS, 1, CHUNK)
    comb = jnp.concatenate([src, dst, att_i], axis=2)
    parts1 = _segsum128(x, comb)
    h1 = _dense(x, parts1, W1_0, b1_0, W2_0, b2_0)
    parts2 = _segsum64(h1, comb)
    out = _dense(h1, parts2, W1_1, b1_1, W2_1, b2_1)
    return out


# speedup vs baseline: 8.1870x; 1.0082x over previous
import functools

import jax
import jax.numpy as jnp
from jax import lax
from jax.experimental import pallas as pl
from jax.experimental.pallas import tpu as pltpu
from jax.experimental.pallas import tpu_sc as plsc

N = 10000
E = 320000
NC, NS, L = 2, 16, 16
NW = NC * NS
EPW = E // NW
CHUNK = 80
NCHUNKS = EPW // CHUNK
ROWS_PER_TILE = N // NS
_BLOCKS = [(k * 80, 80) for k in range(7)] + [(560, 65)]


def _make_segsum(D):
    JV = D // L
    mesh = plsc.VectorSubcoreMesh(core_axis_name="c", subcore_axis_name="s")

    @functools.partial(
        pl.kernel,
        out_type=jax.ShapeDtypeStruct((NC, N, D), jnp.float32),
        mesh=mesh,
        compiler_params=pltpu.CompilerParams(use_tc_tiling_on_sc=False),
        scratch_types=[
            pltpu.VMEM((8, 3, CHUNK), jnp.int32),
            pltpu.VMEM((4, CHUNK, D), jnp.float32),
            pltpu.VMEM_SHARED((N, D), jnp.float32),
            [pltpu.SemaphoreType.DMA] * 4,
            [pltpu.SemaphoreType.DMA] * 4,
            [pltpu.SemaphoreType.DMA] * 8,
        ],
    )
    def seg(h_hbm, comb_hbm, out_hbm,
            ring_v, rows_v, acc_sh, gsem, ssem, csem):
        c = lax.axis_index("c")
        s = lax.axis_index("s")
        wid = s * NC + c

        def start_comb(i, cs):
            pltpu.async_copy(comb_hbm.at[wid, i], ring_v.at[cs], csem[cs])

        def wait_comb(i, cs):
            pltpu.make_async_copy(comb_hbm.at[wid, i], ring_v.at[cs],
                                  csem[cs]).wait()

        def start_gather(cs, rs):
            pltpu.async_copy(h_hbm.at[ring_v.at[cs, 0]], rows_v.at[rs],
                             gsem[rs])

        def wait_gather(cs, rs):
            pltpu.make_async_copy(h_hbm.at[ring_v.at[cs, 0]], rows_v.at[rs],
                                  gsem[rs]).wait()

        def start_scat(cs, rs):
            pltpu.async_copy(rows_v.at[rs], acc_sh.at[ring_v.at[cs, 1]],
                             ssem[rs], add=True)

        def wait_scat(cs, rs):
            pltpu.make_async_copy(rows_v.at[rs], acc_sh.at[ring_v.at[cs, 1]],
                                  ssem[rs]).wait()

        def scale(i, cs, rs):
            def group_body(g, _):
                av = lax.bitcast_convert_type(
                    ring_v[cs, 2, pl.ds(g * L, L)], jnp.float32)
                for e16 in range(L):
                    a = av[e16]
                    rr = g * L + e16
                    for j in range(JV):
                        sl = pl.ds(j * L, L)
                        rows_v[rs, rr, sl] = rows_v[rs, rr, sl] * a
                return 0
            lax.fori_loop(0, CHUNK // L, group_body, 0)

        for k in range(6):
            start_comb(k, k)
        wait_comb(0, 0)
        start_gather(0, 0)
        wait_comb(1, 1)
        start_gather(1, 1)

        def zero_row(r, _):
            for j in range(JV):
                rows_v[3, r, pl.ds(j * L, L)] = jnp.zeros((L,), jnp.float32)
            return 0
        lax.fori_loop(0, CHUNK, zero_row, 0)
        for r0, nr in _BLOCKS:
            pltpu.sync_copy(
                rows_v.at[3, pl.ds(0, nr)],
                acc_sh.at[pl.ds(s * ROWS_PER_TILE + r0, nr)])
        plsc.subcore_barrier()

        def octet_body(k, _):
            for p in range(8):
                i = 8 * k + p
                rs = p & 3

                @pl.when(i <= NCHUNKS - 1)
                def _():
                    wait_gather(p, rs)

                @pl.when(jnp.logical_and(i >= 2, i <= NCHUNKS + 1))
                def _():
                    wait_scat((p + 6) % 8, (p + 2) % 4)

                @pl.when(i <= NCHUNKS - 7)
                def _():
                    start_comb(i + 6, (p + 6) % 8)

                @pl.when(i <= NCHUNKS - 3)
                def _():
                    wait_comb(i + 2, (p + 2) % 8)
                    start_gather((p + 2) % 8, (p + 2) % 4)

                @pl.when(i <= NCHUNKS - 1)
                def _():
                    scale(i, p, rs)
                    start_scat(p, rs)
            return 0
        lax.fori_loop(0, (NCHUNKS + 9) // 8, octet_body, 0)

        plsc.subcore_barrier()

        a0 = s * ROWS_PER_TILE
        pltpu.sync_copy(acc_sh.at[pl.ds(a0, ROWS_PER_TILE)],
                        out_hbm.at[c, pl.ds(a0, ROWS_PER_TILE)])

    return seg


_segsum128 = _make_segsum(128)
_segsum64 = _make_segsum(64)

_DENSE_BLK = 2000


def _dense_body(h_ref, p0_ref, p1_ref, w1_ref, b1_ref, w2_ref, b2_ref, o_ref):
    h = h_ref[...]
    nh = p0_ref[0] + p1_ref[0]
    z1 = jnp.dot(h + nh, w1_ref[...],
                 preferred_element_type=jnp.float32) + b1_ref[...]
    z2 = jnp.dot(h * nh, w2_ref[...],
                 preferred_element_type=jnp.float32) + b2_ref[...]
    o_ref[...] = (jnp.where(z1 >= 0, z1, 0.01 * z1)
                  + jnp.where(z2 >= 0, z2, 0.01 * z2))


def _dense(h, parts, w1, b1, w2, b2):
    din = h.shape[1]
    dout = w1.shape[1]
    b1 = b1.reshape(1, dout)
    b2 = b2.reshape(1, dout)
    return pl.pallas_call(
        _dense_body,
        grid=(N // _DENSE_BLK,),
        in_specs=[
            pl.BlockSpec((_DENSE_BLK, din), lambda i: (i, 0)),
            pl.BlockSpec((1, _DENSE_BLK, din), lambda i: (0, i, 0)),
            pl.BlockSpec((1, _DENSE_BLK, din), lambda i: (1, i, 0)),
            pl.BlockSpec((din, dout), lambda i: (0, 0)),
            pl.BlockSpec((1, dout), lambda i: (0, 0)),
            pl.BlockSpec((din, dout), lambda i: (0, 0)),
            pl.BlockSpec((1, dout), lambda i: (0, 0)),
        ],
        out_specs=pl.BlockSpec((_DENSE_BLK, dout), lambda i: (i, 0)),
        out_shape=jax.ShapeDtypeStruct((N, dout), jnp.float32),
    )(h, parts, parts, w1, b1, w2, b2)


def kernel(x, edge_index, att, W1_0, b1_0, W2_0, b2_0, W1_1, b1_1, W2_1, b2_1):
    src = edge_index[0].reshape(NW, NCHUNKS, 1, CHUNK)
    dst = edge_index[1].reshape(NW, NCHUNKS, 1, CHUNK)
    att_i = lax.bitcast_convert_type(att, jnp.int32).reshape(
        NW, NCHUNKS, 1, CHUNK)
    comb = jnp.concatenate([src, dst, att_i], axis=2)
    parts1 = _segsum128(x, comb)
    h1 = _dense(x, parts1, W1_0, b1_0, W2_0, b2_0)
    parts2 = _segsum64(h1, comb)
    out = _dense(h1, parts2, W1_1, b1_1, W2_1, b2_1)
    return out
